# baseline (speedup 1.0000x reference)
import functools
import math

import jax
import jax.numpy as jnp
from jax import lax
from jax.experimental import pallas as pl
from jax.experimental.pallas import tpu as pltpu
from jax.experimental.pallas import tpu_sc as plsc

N = 10000
D = 128
DE = 4
H = 128
E = 160000

NP = 10240
EP = 163840
CH = 128
HH = H // 2
EDGES_PER_TILE = EP // 16
CHUNKS_PER_TILE = EDGES_PER_TILE // CH
ROWS_PER_TILE = (NP // 2) // 16
NB = 256
GRID_N = NP // NB

_MSG_SCALE = 1.0 / math.sqrt(D * DE)
_UPD_SCALE = 1.0 / math.sqrt(D * H)


def _silu(v):
    return v / (1.0 + jnp.exp(-v))


def _y_body(x_ref, wm_ref, y_ref):
    y_ref[...] = jnp.dot(x_ref[...], wm_ref[...],
                         preferred_element_type=jnp.float32) * _MSG_SCALE


def _y_call(x, wmr):
    return pl.pallas_call(
        _y_body,
        grid=(GRID_N,),
        in_specs=[pl.BlockSpec((NB, D), lambda i: (i, 0)),
                  pl.BlockSpec((D, DE * H), lambda i: (0, 0))],
        out_specs=pl.BlockSpec((NB, DE * H), lambda i: (i, 0)),
        out_shape=jax.ShapeDtypeStruct((NP, DE * H), jnp.float32),
    )(x, wmr)


def _update_core(x_ref, agg_ref, wu_ref):
    xb = x_ref[...]
    ab = jnp.concatenate([agg_ref[0], agg_ref[1]], axis=-1)
    t = jnp.dot(xb, wu_ref[...], preferred_element_type=jnp.float32)
    acc = ab[:, 0:1] * t[:, 0:D]
    for j in range(1, H):
        acc = acc + ab[:, j:j + 1] * t[:, j * D:(j + 1) * D]
    return xb + _silu(acc * _UPD_SCALE)


def _update_y_body(x_ref, agg_ref, wu_ref, wm_ref, xo_ref, yo_ref):
    xn = _update_core(x_ref, agg_ref, wu_ref)
    xo_ref[...] = xn
    yo_ref[...] = jnp.dot(xn, wm_ref[...],
                          preferred_element_type=jnp.float32) * _MSG_SCALE


def _update_body(x_ref, agg_ref, wu_ref, xo_ref):
    xo_ref[...] = _update_core(x_ref, agg_ref, wu_ref)


def _update_y_call(x, agg, wur, wmr):
    return pl.pallas_call(
        _update_y_body,
        grid=(GRID_N,),
        in_specs=[pl.BlockSpec((NB, D), lambda i: (i, 0)),
                  pl.BlockSpec((2, NB, HH), lambda i: (0, i, 0)),
                  pl.BlockSpec((D, H * D), lambda i: (0, 0)),
                  pl.BlockSpec((D, DE * H), lambda i: (0, 0))],
        out_specs=[pl.BlockSpec((NB, D), lambda i: (i, 0)),
                   pl.BlockSpec((NB, DE * H), lambda i: (i, 0))],
        out_shape=[jax.ShapeDtypeStruct((NP, D), jnp.float32),
                   jax.ShapeDtypeStruct((NP, DE * H), jnp.float32)],
    )(x, agg, wur, wmr)


def _update_call(x, agg, wur):
    return pl.pallas_call(
        _update_body,
        grid=(GRID_N,),
        in_specs=[pl.BlockSpec((NB, D), lambda i: (i, 0)),
                  pl.BlockSpec((2, NB, HH), lambda i: (0, i, 0)),
                  pl.BlockSpec((D, H * D), lambda i: (0, 0))],
        out_specs=pl.BlockSpec((NB, D), lambda i: (i, 0)),
        out_shape=jax.ShapeDtypeStruct((NP, D), jnp.float32),
    )(x, agg, wur)


_sc_mesh = plsc.VectorSubcoreMesh(core_axis_name="c", subcore_axis_name="s")


@functools.partial(
    pl.kernel,
    out_type=jax.ShapeDtypeStruct((NP, D), jnp.float32),
    mesh=_sc_mesh,
    scratch_types=[
        pltpu.VMEM((CH,), jnp.int32),
        pltpu.VMEM((CH,), jnp.int32),
        pltpu.VMEM((CH + 16,), jnp.int32),
        pltpu.VMEM((CH,), jnp.int32),
        pltpu.VMEM((CH * DE + 16,), jnp.float32),
        pltpu.VMEM((CH, DE * HH), jnp.float32),
        pltpu.VMEM((CH, D), jnp.float32),
        pltpu.VMEM_SHARED((NP // 2, D), jnp.float32),
        pltpu.SemaphoreType.DMA,
    ],
)
def _edge_kernel(y_hbm, col_hbm, row_hbm, ea_hbm, z_hbm, out_hbm,
                 colv, gatv, rowv, rowv2, eav, yv, mv, aggsh, sem):
    core = lax.axis_index("c")
    sub = lax.axis_index("s")

    pltpu.sync_copy(z_hbm, aggsh.at[pl.ds(sub * ROWS_PER_TILE, ROWS_PER_TILE)])
    plsc.subcore_barrier()

    def chunk_body(c, carry):
        base = pl.multiple_of(sub * EDGES_PER_TILE + c * CH, CH)
        pltpu.sync_copy(col_hbm.at[pl.ds(base, CH)], colv)
        for i in range(CH // 16):
            o = i * 16
            gatv[pl.ds(o, 16)] = colv[pl.ds(o, 16)] * 2 + core
        gather = pltpu.async_copy(y_hbm.at[gatv], yv, sem)
        pltpu.sync_copy(row_hbm.at[pl.ds(base, CH)], rowv.at[pl.ds(0, CH)])
        pltpu.sync_copy(ea_hbm.at[pl.ds(base * DE, CH * DE)],
                        eav.at[pl.ds(0, CH * DE)])
        for i in range(CH // 16):
            o = i * 16
            rowv2[pl.ds(o, 16)] = lax.shift_right_logical(
                rowv[pl.ds(o, 16)], 1)
        gather.wait()

        def edge_body(e, carry2):
            evec = eav[pl.ds(e * DE, 16)]
            a0 = evec[0]
            a1 = evec[1]
            a2 = evec[2]
            a3 = evec[3]
            rvec = rowv[pl.ds(e, 16)]
            p64 = (rvec[0] & 1) * HH
            q64 = HH - p64
            for k in range(HH // 16):
                o = k * 16
                v = (a0 * yv[e, pl.ds(o, 16)]
                     + a1 * yv[e, pl.ds(HH + o, 16)]
                     + a2 * yv[e, pl.ds(2 * HH + o, 16)]
                     + a3 * yv[e, pl.ds(3 * HH + o, 16)])
                mv[e, pl.ds(p64 + o, 16)] = _silu(v)
                mv[e, pl.ds(q64 + o, 16)] = jnp.zeros((16,), jnp.float32)
            return carry2

        lax.fori_loop(0, CH, edge_body, 0)
        pltpu.sync_copy(mv, aggsh.at[rowv2], add=True)
        return carry

    lax.fori_loop(0, CHUNKS_PER_TILE, chunk_body, 0)
    plsc.subcore_barrier()
    off = sub * ROWS_PER_TILE
    pltpu.sync_copy(aggsh.at[pl.ds(off, ROWS_PER_TILE)],
                    out_hbm.at[pl.ds(core * (NP // 2) + off, ROWS_PER_TILE)])


def _rearrange_wm(Wm):
    return Wm.reshape(D, DE, 2, HH).transpose(0, 2, 1, 3).reshape(D, DE * H)


def kernel(node_features, edge_index, edge_attr, Wm1, Wu1, Wm2, Wu2):
    f32 = jnp.float32
    i32 = jnp.int32
    x0 = jnp.concatenate([node_features, jnp.zeros((NP - N, D), f32)])
    colp = jnp.concatenate([edge_index[1], jnp.zeros((EP - E,), i32)])
    rowp = jnp.concatenate([edge_index[0], jnp.zeros((EP - E,), i32)])
    eap = jnp.concatenate([edge_attr,
                           jnp.zeros((EP - E, DE), f32)]).reshape(EP * DE)
    z = jnp.zeros((ROWS_PER_TILE, D), f32)
    wm1r = _rearrange_wm(Wm1)
    wm2r = _rearrange_wm(Wm2)
    wu1r = Wu1.reshape(D, H * D)
    wu2r = Wu2.reshape(D, H * D)

    def _unpack(out):
        return out.reshape(2, NP, HH)

    y1 = _y_call(x0, wm1r).reshape(2 * NP, DE * HH)
    agg1 = _unpack(_edge_kernel(y1, colp, rowp, eap, z))
    x1, y2 = _update_y_call(x0, agg1, wu1r, wm2r)
    agg2 = _unpack(_edge_kernel(y2.reshape(2 * NP, DE * HH), colp, rowp,
                                eap, z))
    x2 = _update_call(x1, agg2, wu2r)
    return x2[:N]


# speedup vs baseline: 1.0003x; 1.0003x over previous
import functools
import math

import jax
import jax.numpy as jnp
from jax import lax
from jax.experimental import pallas as pl
from jax.experimental.pallas import tpu as pltpu
from jax.experimental.pallas import tpu_sc as plsc

N = 10000
D = 128
DE = 4
H = 128
E = 160000

NP = 10240
EP = 163840
CH = 128
HH = H // 2
EDGES_PER_TILE = EP // 16
CHUNKS_PER_TILE = EDGES_PER_TILE // CH
ROWS_PER_TILE = (NP // 2) // 16
NB = 256
GRID_N = NP // NB

_MSG_SCALE = 1.0 / math.sqrt(D * DE)
_UPD_SCALE = 1.0 / math.sqrt(D * H)


def _silu(v):
    return v / (1.0 + jnp.exp(-v))


def _y_body(x_ref, wm_ref, y_ref):
    y_ref[...] = jnp.dot(x_ref[...], wm_ref[...],
                         preferred_element_type=jnp.float32) * _MSG_SCALE


def _y_call(x, wmr):
    return pl.pallas_call(
        _y_body,
        grid=(GRID_N,),
        in_specs=[pl.BlockSpec((NB, D), lambda i: (i, 0)),
                  pl.BlockSpec((D, DE * H), lambda i: (0, 0))],
        out_specs=pl.BlockSpec((NB, DE * H), lambda i: (i, 0)),
        out_shape=jax.ShapeDtypeStruct((NP, DE * H), jnp.float32),
    )(x, wmr)


def _update_core(x_ref, agg_ref, wu_ref):
    xb = x_ref[...]
    ab = jnp.concatenate([agg_ref[0], agg_ref[1]], axis=-1)
    t = jnp.dot(xb, wu_ref[...], preferred_element_type=jnp.float32)
    acc = ab[:, 0:1] * t[:, 0:D]
    for j in range(1, H):
        acc = acc + ab[:, j:j + 1] * t[:, j * D:(j + 1) * D]
    return xb + _silu(acc * _UPD_SCALE)


def _update_y_body(x_ref, agg_ref, wu_ref, wm_ref, xo_ref, yo_ref):
    xn = _update_core(x_ref, agg_ref, wu_ref)
    xo_ref[...] = xn
    yo_ref[...] = jnp.dot(xn, wm_ref[...],
                          preferred_element_type=jnp.float32) * _MSG_SCALE


def _update_body(x_ref, agg_ref, wu_ref, xo_ref):
    xo_ref[...] = _update_core(x_ref, agg_ref, wu_ref)


def _update_y_call(x, agg, wur, wmr):
    return pl.pallas_call(
        _update_y_body,
        grid=(GRID_N,),
        in_specs=[pl.BlockSpec((NB, D), lambda i: (i, 0)),
                  pl.BlockSpec((2, NB, HH), lambda i: (0, i, 0)),
                  pl.BlockSpec((D, H * D), lambda i: (0, 0)),
                  pl.BlockSpec((D, DE * H), lambda i: (0, 0))],
        out_specs=[pl.BlockSpec((NB, D), lambda i: (i, 0)),
                   pl.BlockSpec((NB, DE * H), lambda i: (i, 0))],
        out_shape=[jax.ShapeDtypeStruct((NP, D), jnp.float32),
                   jax.ShapeDtypeStruct((NP, DE * H), jnp.float32)],
    )(x, agg, wur, wmr)


def _update_call(x, agg, wur):
    return pl.pallas_call(
        _update_body,
        grid=(GRID_N,),
        in_specs=[pl.BlockSpec((NB, D), lambda i: (i, 0)),
                  pl.BlockSpec((2, NB, HH), lambda i: (0, i, 0)),
                  pl.BlockSpec((D, H * D), lambda i: (0, 0))],
        out_specs=pl.BlockSpec((NB, D), lambda i: (i, 0)),
        out_shape=jax.ShapeDtypeStruct((NP, D), jnp.float32),
    )(x, agg, wur)


_sc_mesh = plsc.VectorSubcoreMesh(core_axis_name="c", subcore_axis_name="s")


@functools.partial(
    pl.kernel,
    out_type=jax.ShapeDtypeStruct((NP, D), jnp.float32),
    mesh=_sc_mesh,
    scratch_types=[
        pltpu.VMEM((CH,), jnp.int32),
        pltpu.VMEM((CH,), jnp.int32),
        pltpu.VMEM((CH + 16,), jnp.int32),
        pltpu.VMEM((CH,), jnp.int32),
        pltpu.VMEM((CH * DE + 16,), jnp.float32),
        pltpu.VMEM((CH, DE * HH), jnp.float32),
        pltpu.VMEM((CH, D), jnp.float32),
        pltpu.VMEM_SHARED((NP // 2, D), jnp.float32),
        pltpu.SemaphoreType.DMA,
    ],
)
def _edge_kernel(y_hbm, col_hbm, row_hbm, ea_hbm, z_hbm, out_hbm,
                 colv, gatv, rowv, rowv2, eav, yv, mv, aggsh, sem):
    core = lax.axis_index("c")
    sub = lax.axis_index("s")

    pltpu.sync_copy(z_hbm, aggsh.at[pl.ds(sub * ROWS_PER_TILE, ROWS_PER_TILE)])
    plsc.subcore_barrier()

    def chunk_body(c, carry):
        base = pl.multiple_of(sub * EDGES_PER_TILE + c * CH, CH)
        pltpu.sync_copy(col_hbm.at[pl.ds(base, CH)], colv)
        for i in range(CH // 16):
            o = i * 16
            gatv[pl.ds(o, 16)] = colv[pl.ds(o, 16)] * 2 + core
        gather = pltpu.async_copy(y_hbm.at[gatv], yv, sem)
        pltpu.sync_copy(row_hbm.at[pl.ds(base, CH)], rowv.at[pl.ds(0, CH)])
        pltpu.sync_copy(ea_hbm.at[pl.ds(base * DE, CH * DE)],
                        eav.at[pl.ds(0, CH * DE)])
        for i in range(CH // 16):
            o = i * 16
            rowv2[pl.ds(o, 16)] = lax.shift_right_logical(
                rowv[pl.ds(o, 16)], 1)
        gather.wait()

        def edge_body(e, carry2):
            evec = eav[pl.ds(e * DE, 16)]
            a0 = evec[0]
            a1 = evec[1]
            a2 = evec[2]
            a3 = evec[3]
            rvec = rowv[pl.ds(e, 16)]
            p64 = (rvec[0] & 1) * HH
            q64 = HH - p64
            for k in range(HH // 16):
                o = k * 16
                v = (a0 * yv[e, pl.ds(o, 16)]
                     + a1 * yv[e, pl.ds(HH + o, 16)]
                     + a2 * yv[e, pl.ds(2 * HH + o, 16)]
                     + a3 * yv[e, pl.ds(3 * HH + o, 16)])
                mv[e, pl.ds(p64 + o, 16)] = _silu(v)
                mv[e, pl.ds(q64 + o, 16)] = jnp.zeros((16,), jnp.float32)
            return carry2

        lax.fori_loop(0, CH, edge_body, 0, unroll=4)
        pltpu.sync_copy(mv, aggsh.at[rowv2], add=True)
        return carry

    lax.fori_loop(0, CHUNKS_PER_TILE, chunk_body, 0)
    plsc.subcore_barrier()
    off = sub * ROWS_PER_TILE
    pltpu.sync_copy(aggsh.at[pl.ds(off, ROWS_PER_TILE)],
                    out_hbm.at[pl.ds(core * (NP // 2) + off, ROWS_PER_TILE)])


def _rearrange_wm(Wm):
    return Wm.reshape(D, DE, 2, HH).transpose(0, 2, 1, 3).reshape(D, DE * H)


def kernel(node_features, edge_index, edge_attr, Wm1, Wu1, Wm2, Wu2):
    f32 = jnp.float32
    i32 = jnp.int32
    x0 = jnp.concatenate([node_features, jnp.zeros((NP - N, D), f32)])
    colp = jnp.concatenate([edge_index[1], jnp.zeros((EP - E,), i32)])
    rowp = jnp.concatenate([edge_index[0], jnp.zeros((EP - E,), i32)])
    eap = jnp.concatenate([edge_attr,
                           jnp.zeros((EP - E, DE), f32)]).reshape(EP * DE)
    z = jnp.zeros((ROWS_PER_TILE, D), f32)
    wm1r = _rearrange_wm(Wm1)
    wm2r = _rearrange_wm(Wm2)
    wu1r = Wu1.reshape(D, H * D)
    wu2r = Wu2.reshape(D, H * D)

    def _unpack(out):
        return out.reshape(2, NP, HH)

    y1 = _y_call(x0, wm1r).reshape(2 * NP, DE * HH)
    agg1 = _unpack(_edge_kernel(y1, colp, rowp, eap, z))
    x1, y2 = _update_y_call(x0, agg1, wu1r, wm2r)
    agg2 = _unpack(_edge_kernel(y2.reshape(2 * NP, DE * HH), colp, rowp,
                                eap, z))
    x2 = _update_call(x1, agg2, wu2r)
    return x2[:N]


# speedup vs baseline: 1.0982x; 1.0978x over previous
import functools
import math

import jax
import jax.numpy as jnp
from jax import lax
from jax.experimental import pallas as pl
from jax.experimental.pallas import tpu as pltpu
from jax.experimental.pallas import tpu_sc as plsc

N = 10000
D = 128
DE = 4
H = 128
E = 160000

NP = 10240
EP = 163840
CH = 128
HH = H // 2
EDGES_PER_TILE = EP // 16
CHUNKS_PER_TILE = EDGES_PER_TILE // CH
ROWS_PER_TILE = (NP // 2) // 16
NB = 256
GRID_N = NP // NB

_MSG_SCALE = 1.0 / math.sqrt(D * DE)
_UPD_SCALE = 1.0 / math.sqrt(D * H)


def _silu(v):
    return v / (1.0 + jnp.exp(-v))


def _y_body(x_ref, wm_ref, y_ref):
    y_ref[...] = jnp.dot(x_ref[...], wm_ref[...],
                         preferred_element_type=jnp.float32) * _MSG_SCALE


def _y_call(x, wmr):
    return pl.pallas_call(
        _y_body,
        grid=(GRID_N,),
        in_specs=[pl.BlockSpec((NB, D), lambda i: (i, 0)),
                  pl.BlockSpec((D, DE * H), lambda i: (0, 0))],
        out_specs=pl.BlockSpec((NB, DE * H), lambda i: (i, 0)),
        out_shape=jax.ShapeDtypeStruct((NP, DE * H), jnp.float32),
    )(x, wmr)


def _update_core(x_ref, agg_ref, wu_ref):
    xb = x_ref[...]
    ab = jnp.concatenate([agg_ref[0], agg_ref[1]], axis=-1)
    t = jnp.dot(xb, wu_ref[...], preferred_element_type=jnp.float32)
    acc = ab[:, 0:1] * t[:, 0:D]
    for j in range(1, H):
        acc = acc + ab[:, j:j + 1] * t[:, j * D:(j + 1) * D]
    return xb + _silu(acc * _UPD_SCALE)


def _update_y_body(x_ref, agg_ref, wu_ref, wm_ref, xo_ref, yo_ref):
    xn = _update_core(x_ref, agg_ref, wu_ref)
    xo_ref[...] = xn
    yo_ref[...] = jnp.dot(xn, wm_ref[...],
                          preferred_element_type=jnp.float32) * _MSG_SCALE


def _update_body(x_ref, agg_ref, wu_ref, xo_ref):
    xo_ref[...] = _update_core(x_ref, agg_ref, wu_ref)


def _update_y_call(x, agg, wur, wmr):
    return pl.pallas_call(
        _update_y_body,
        grid=(GRID_N,),
        in_specs=[pl.BlockSpec((NB, D), lambda i: (i, 0)),
                  pl.BlockSpec((2, NB, HH), lambda i: (0, i, 0)),
                  pl.BlockSpec((D, H * D), lambda i: (0, 0)),
                  pl.BlockSpec((D, DE * H), lambda i: (0, 0))],
        out_specs=[pl.BlockSpec((NB, D), lambda i: (i, 0)),
                   pl.BlockSpec((NB, DE * H), lambda i: (i, 0))],
        out_shape=[jax.ShapeDtypeStruct((NP, D), jnp.float32),
                   jax.ShapeDtypeStruct((NP, DE * H), jnp.float32)],
    )(x, agg, wur, wmr)


def _update_call(x, agg, wur):
    return pl.pallas_call(
        _update_body,
        grid=(GRID_N,),
        in_specs=[pl.BlockSpec((NB, D), lambda i: (i, 0)),
                  pl.BlockSpec((2, NB, HH), lambda i: (0, i, 0)),
                  pl.BlockSpec((D, H * D), lambda i: (0, 0))],
        out_specs=pl.BlockSpec((NB, D), lambda i: (i, 0)),
        out_shape=jax.ShapeDtypeStruct((NP, D), jnp.float32),
    )(x, agg, wur)


_sc_mesh = plsc.VectorSubcoreMesh(core_axis_name="c", subcore_axis_name="s")


@functools.partial(
    pl.kernel,
    out_type=jax.ShapeDtypeStruct((NP, D), jnp.float32),
    mesh=_sc_mesh,
    scratch_types=[
        pltpu.VMEM((CH,), jnp.int32),
        pltpu.VMEM((CH,), jnp.int32),
        pltpu.VMEM((CH + 16,), jnp.int32),
        pltpu.VMEM((CH,), jnp.int32),
        pltpu.VMEM((CH * DE + 16,), jnp.float32),
        pltpu.VMEM((CH, DE * HH), jnp.float32),
        pltpu.VMEM((CH, D), jnp.float32),
        pltpu.VMEM_SHARED((NP // 2, D), jnp.float32),
        pltpu.SemaphoreType.DMA,
    ],
)
def _edge_kernel(y_hbm, col_hbm, row_hbm, ea_hbm, z_hbm, out_hbm,
                 colv, gatv, rowv, rowv2, eav, yv, mv, aggsh, sem):
    core = lax.axis_index("c")
    sub = lax.axis_index("s")

    pltpu.sync_copy(z_hbm, aggsh.at[pl.ds(sub * ROWS_PER_TILE, ROWS_PER_TILE)])
    plsc.subcore_barrier()

    def chunk_body(c, carry):
        base = pl.multiple_of(sub * EDGES_PER_TILE + c * CH, CH)
        pltpu.sync_copy(col_hbm.at[pl.ds(base, CH)], colv)
        for i in range(CH // 16):
            o = i * 16
            gatv[pl.ds(o, 16)] = colv[pl.ds(o, 16)] * 2 + core
        gather = pltpu.async_copy(y_hbm.at[gatv], yv, sem)
        pltpu.sync_copy(row_hbm.at[pl.ds(base, CH)], rowv.at[pl.ds(0, CH)])
        pltpu.sync_copy(ea_hbm.at[pl.ds(base * DE, CH * DE)],
                        eav.at[pl.ds(0, CH * DE)])
        for i in range(CH // 16):
            o = i * 16
            rowv2[pl.ds(o, 16)] = lax.shift_right_logical(
                rowv[pl.ds(o, 16)], 1)
        gather.wait()

        def edge_body(e):
            evec = eav[pl.ds(e * DE, 16)]
            a0 = evec[0]
            a1 = evec[1]
            a2 = evec[2]
            a3 = evec[3]
            rvec = rowv[pl.ds(e, 16)]
            p64 = (rvec[0] & 1) * HH
            q64 = HH - p64
            for k in range(HH // 16):
                o = k * 16
                v = (a0 * yv[e, pl.ds(o, 16)]
                     + a1 * yv[e, pl.ds(HH + o, 16)]
                     + a2 * yv[e, pl.ds(2 * HH + o, 16)]
                     + a3 * yv[e, pl.ds(3 * HH + o, 16)])
                mv[e, pl.ds(p64 + o, 16)] = _silu(v)
                mv[e, pl.ds(q64 + o, 16)] = jnp.zeros((16,), jnp.float32)

        plsc.parallel_loop(0, CH, step=1, unroll=8)(edge_body)
        pltpu.sync_copy(mv, aggsh.at[rowv2], add=True)
        return carry

    lax.fori_loop(0, CHUNKS_PER_TILE, chunk_body, 0)
    plsc.subcore_barrier()
    off = sub * ROWS_PER_TILE
    pltpu.sync_copy(aggsh.at[pl.ds(off, ROWS_PER_TILE)],
                    out_hbm.at[pl.ds(core * (NP // 2) + off, ROWS_PER_TILE)])


def _rearrange_wm(Wm):
    return Wm.reshape(D, DE, 2, HH).transpose(0, 2, 1, 3).reshape(D, DE * H)


def kernel(node_features, edge_index, edge_attr, Wm1, Wu1, Wm2, Wu2):
    f32 = jnp.float32
    i32 = jnp.int32
    x0 = jnp.concatenate([node_features, jnp.zeros((NP - N, D), f32)])
    colp = jnp.concatenate([edge_index[1], jnp.zeros((EP - E,), i32)])
    rowp = jnp.concatenate([edge_index[0], jnp.zeros((EP - E,), i32)])
    eap = jnp.concatenate([edge_attr,
                           jnp.zeros((EP - E, DE), f32)]).reshape(EP * DE)
    z = jnp.zeros((ROWS_PER_TILE, D), f32)
    wm1r = _rearrange_wm(Wm1)
    wm2r = _rearrange_wm(Wm2)
    wu1r = Wu1.reshape(D, H * D)
    wu2r = Wu2.reshape(D, H * D)

    def _unpack(out):
        return out.reshape(2, NP, HH)

    y1 = _y_call(x0, wm1r).reshape(2 * NP, DE * HH)
    agg1 = _unpack(_edge_kernel(y1, colp, rowp, eap, z))
    x1, y2 = _update_y_call(x0, agg1, wu1r, wm2r)
    agg2 = _unpack(_edge_kernel(y2.reshape(2 * NP, DE * HH), colp, rowp,
                                eap, z))
    x2 = _update_call(x1, agg2, wu2r)
    return x2[:N]


# speedup vs baseline: 1.3604x; 1.2388x over previous
import functools
import math

import jax
import jax.numpy as jnp
from jax import lax
from jax.experimental import pallas as pl
from jax.experimental.pallas import tpu as pltpu
from jax.experimental.pallas import tpu_sc as plsc

N = 10000
D = 128
DE = 4
H = 128
E = 160000

NP = 10240
EP = 163840
CH = 128
HH = H // 2
EDGES_PER_TILE = EP // 16
CHUNKS_PER_TILE = EDGES_PER_TILE // CH
ROWS_PER_TILE = (NP // 2) // 16
NB = 256
GRID_N = NP // NB

_MSG_SCALE = 1.0 / math.sqrt(D * DE)
_UPD_SCALE = 1.0 / math.sqrt(D * H)


def _silu(v):
    return v / (1.0 + jnp.exp(-v))


def _y_body(x_ref, wm_ref, y_ref):
    y_ref[...] = jnp.dot(x_ref[...], wm_ref[...],
                         preferred_element_type=jnp.float32) * _MSG_SCALE


def _y_call(x, wmr):
    return pl.pallas_call(
        _y_body,
        grid=(GRID_N,),
        in_specs=[pl.BlockSpec((NB, D), lambda i: (i, 0)),
                  pl.BlockSpec((D, DE * H), lambda i: (0, 0))],
        out_specs=pl.BlockSpec((NB, DE * H), lambda i: (i, 0)),
        out_shape=jax.ShapeDtypeStruct((NP, DE * H), jnp.float32),
    )(x, wmr)


def _update_core(x_ref, agg_ref, wu_ref):
    xb = x_ref[...]
    ab = jnp.concatenate([agg_ref[0], agg_ref[1]], axis=-1)
    t = jnp.dot(xb, wu_ref[...], preferred_element_type=jnp.float32)
    acc = ab[:, 0:1] * t[:, 0:D]
    for j in range(1, H):
        acc = acc + ab[:, j:j + 1] * t[:, j * D:(j + 1) * D]
    return xb + _silu(acc * _UPD_SCALE)


def _update_y_body(x_ref, agg_ref, wu_ref, wm_ref, xo_ref, yo_ref):
    xn = _update_core(x_ref, agg_ref, wu_ref)
    xo_ref[...] = xn
    yo_ref[...] = jnp.dot(xn, wm_ref[...],
                          preferred_element_type=jnp.float32) * _MSG_SCALE


def _update_body(x_ref, agg_ref, wu_ref, xo_ref):
    xo_ref[...] = _update_core(x_ref, agg_ref, wu_ref)


def _update_y_call(x, agg, wur, wmr):
    return pl.pallas_call(
        _update_y_body,
        grid=(GRID_N,),
        in_specs=[pl.BlockSpec((NB, D), lambda i: (i, 0)),
                  pl.BlockSpec((2, NB, HH), lambda i: (0, i, 0)),
                  pl.BlockSpec((D, H * D), lambda i: (0, 0)),
                  pl.BlockSpec((D, DE * H), lambda i: (0, 0))],
        out_specs=[pl.BlockSpec((NB, D), lambda i: (i, 0)),
                   pl.BlockSpec((NB, DE * H), lambda i: (i, 0))],
        out_shape=[jax.ShapeDtypeStruct((NP, D), jnp.float32),
                   jax.ShapeDtypeStruct((NP, DE * H), jnp.float32)],
    )(x, agg, wur, wmr)


def _update_call(x, agg, wur):
    return pl.pallas_call(
        _update_body,
        grid=(GRID_N,),
        in_specs=[pl.BlockSpec((NB, D), lambda i: (i, 0)),
                  pl.BlockSpec((2, NB, HH), lambda i: (0, i, 0)),
                  pl.BlockSpec((D, H * D), lambda i: (0, 0))],
        out_specs=pl.BlockSpec((NB, D), lambda i: (i, 0)),
        out_shape=jax.ShapeDtypeStruct((NP, D), jnp.float32),
    )(x, agg, wur)


_sc_mesh = plsc.VectorSubcoreMesh(core_axis_name="c", subcore_axis_name="s")


PKW = CH + CH + CH * DE


@functools.partial(
    pl.kernel,
    out_type=jax.ShapeDtypeStruct((NP, D), jnp.float32),
    mesh=_sc_mesh,
    scratch_types=[
        pltpu.VMEM((PKW + 16,), jnp.int32),
        pltpu.VMEM((PKW + 16,), jnp.int32),
        pltpu.VMEM((CH,), jnp.int32),
        pltpu.VMEM((CH,), jnp.int32),
        pltpu.VMEM((CH,), jnp.int32),
        pltpu.VMEM((CH,), jnp.int32),
        pltpu.VMEM((CH, DE * HH), jnp.float32),
        pltpu.VMEM((CH, DE * HH), jnp.float32),
        pltpu.VMEM((CH, D), jnp.float32),
        pltpu.VMEM_SHARED((NP // 2, D), jnp.float32),
        pltpu.SemaphoreType.DMA,
        pltpu.SemaphoreType.DMA,
        pltpu.SemaphoreType.DMA,
        pltpu.SemaphoreType.DMA,
    ],
)
def _edge_kernel(y_hbm, pk_hbm, z_hbm, out_hbm,
                 pk0, pk1, gatv0, gatv1, rowv0, rowv1, yv0, yv1, mv, aggsh,
                 semp0, semp1, semg0, semg1):
    core = lax.axis_index("c")
    sub = lax.axis_index("s")
    pks = (pk0, pk1)
    gatvs = (gatv0, gatv1)
    rowvs = (rowv0, rowv1)
    yvs = (yv0, yv1)
    semps = (semp0, semp1)
    semgs = (semg0, semg1)

    pltpu.sync_copy(z_hbm, aggsh.at[pl.ds(sub * ROWS_PER_TILE, ROWS_PER_TILE)])
    plsc.subcore_barrier()

    cbase = sub * CHUNKS_PER_TILE

    def _prep_idx(b, c):
        for i in range(CH // 16):
            o = i * 16
            gatvs[b][pl.ds(o, 16)] = pks[b][pl.ds(o, 16)] * 2 + core
            rowvs[b][pl.ds(o, 16)] = lax.shift_right_logical(
                pks[b][pl.ds(CH + o, 16)], 1)
        pltpu.async_copy(y_hbm.at[gatvs[b]], yvs[b], semgs[b])

    pltpu.sync_copy(pk_hbm.at[pl.ds(cbase * PKW, PKW)], pk0.at[pl.ds(0, PKW)])
    _prep_idx(0, 0)
    pltpu.async_copy(pk_hbm.at[pl.ds((cbase + 1) * PKW, PKW)],
                     pk1.at[pl.ds(0, PKW)], semp1)

    def outer_body(t, carry):
        for b in range(2):
            i = 2 * t + b
            nb = 1 - b

            @pl.when(i + 1 < CHUNKS_PER_TILE)
            def _():
                pltpu.make_async_copy(
                    pk_hbm.at[pl.ds(0, PKW)],
                    pks[nb].at[pl.ds(0, PKW)], semps[nb]).wait()
                _prep_idx(nb, i + 1)

            pltpu.make_async_copy(y_hbm.at[gatvs[b]], yvs[b],
                                  semgs[b]).wait()

            def edge_body(e):
                evec = lax.bitcast_convert_type(
                    pks[b][pl.ds(2 * CH + e * DE, 16)], jnp.float32)
                a0 = evec[0]
                a1 = evec[1]
                a2 = evec[2]
                a3 = evec[3]
                rvec = pks[b][pl.ds(CH + e, 16)]
                p64 = (rvec[0] & 1) * HH
                q64 = HH - p64
                yb = yvs[b]
                for k in range(HH // 16):
                    o = k * 16
                    v = (a0 * yb[e, pl.ds(o, 16)]
                         + a1 * yb[e, pl.ds(HH + o, 16)]
                         + a2 * yb[e, pl.ds(2 * HH + o, 16)]
                         + a3 * yb[e, pl.ds(3 * HH + o, 16)])
                    mv[e, pl.ds(p64 + o, 16)] = _silu(v)
                    mv[e, pl.ds(q64 + o, 16)] = jnp.zeros((16,), jnp.float32)

            plsc.parallel_loop(0, CH, step=1, unroll=8)(edge_body)
            pltpu.sync_copy(mv, aggsh.at[rowvs[b]], add=True)

            @pl.when(i + 2 < CHUNKS_PER_TILE)
            def _():
                pltpu.async_copy(
                    pk_hbm.at[pl.ds((cbase + i + 2) * PKW, PKW)],
                    pks[b].at[pl.ds(0, PKW)], semps[b])
        return carry

    lax.fori_loop(0, CHUNKS_PER_TILE // 2, outer_body, 0)
    plsc.subcore_barrier()
    off = sub * ROWS_PER_TILE
    pltpu.sync_copy(aggsh.at[pl.ds(off, ROWS_PER_TILE)],
                    out_hbm.at[pl.ds(core * (NP // 2) + off, ROWS_PER_TILE)])


def _rearrange_wm(Wm):
    return Wm.reshape(D, DE, 2, HH).transpose(0, 2, 1, 3).reshape(D, DE * H)


def kernel(node_features, edge_index, edge_attr, Wm1, Wu1, Wm2, Wu2):
    f32 = jnp.float32
    i32 = jnp.int32
    x0 = jnp.concatenate([node_features, jnp.zeros((NP - N, D), f32)])
    colp = jnp.concatenate([edge_index[1], jnp.zeros((EP - E,), i32)])
    rowp = jnp.concatenate([edge_index[0], jnp.zeros((EP - E,), i32)])
    eap = jnp.concatenate([edge_attr, jnp.zeros((EP - E, DE), f32)])
    nch = EP // CH
    pk = jnp.concatenate(
        [colp.reshape(nch, CH), rowp.reshape(nch, CH),
         lax.bitcast_convert_type(eap, i32).reshape(nch, CH * DE)],
        axis=1).reshape(nch * PKW)
    z = jnp.zeros((ROWS_PER_TILE, D), f32)
    wm1r = _rearrange_wm(Wm1)
    wm2r = _rearrange_wm(Wm2)
    wu1r = Wu1.reshape(D, H * D)
    wu2r = Wu2.reshape(D, H * D)

    def _unpack(out):
        return out.reshape(2, NP, HH)

    y1 = _y_call(x0, wm1r).reshape(2 * NP, DE * HH)
    agg1 = _unpack(_edge_kernel(y1, pk, z))
    x1, y2 = _update_y_call(x0, agg1, wu1r, wm2r)
    agg2 = _unpack(_edge_kernel(y2.reshape(2 * NP, DE * HH), pk, z))
    x2 = _update_call(x1, agg2, wu2r)
    return x2[:N]
